# baseline (speedup 1.0000x reference)
import functools

import numpy as np
import jax
import jax.numpy as jnp
from jax import lax
from jax.experimental import pallas as pl

_B, _V = 128, 100000
_ROWS = 16
_TEMP = np.float32(0.8)
_TOP_P = np.float32(0.9)
_ONE_BITS = 0x3F800000
_TINY = np.float32(np.finfo(np.float32).tiny)
_VSTAR_LB = np.float32(4.5e-7)
_SEARCH_K = 4
_SEARCH_STEPS = 14


def _threefry2x32(k1, k2, x0, x1):
    ks0 = np.uint32(k1)
    ks1 = np.uint32(k2)
    ks2 = np.uint32(ks0 ^ ks1 ^ np.uint32(0x1BD11BDA))
    rotations = [[13, 15, 26, 6], [17, 29, 16, 24]]
    ks = [ks0, ks1, ks2]
    x0 = (x0 + ks0).astype(np.uint32)
    x1 = (x1 + ks1).astype(np.uint32)
    for i in range(5):
        for r in rotations[i % 2]:
            x0 = (x0 + x1).astype(np.uint32)
            x1 = ((x1 << np.uint32(r)) | (x1 >> np.uint32(32 - r))).astype(np.uint32)
            x1 = (x1 ^ x0).astype(np.uint32)
        x0 = (x0 + ks[(i + 1) % 3]).astype(np.uint32)
        x1 = (x1 + ks[(i + 2) % 3] + np.uint32(i + 1)).astype(np.uint32)
    return x0, x1


@functools.cache
def _gumbel_bits_const():
    n = _B * _V
    c0 = np.zeros(n, dtype=np.uint32)
    c1 = np.arange(n, dtype=np.uint32)
    y0, y1 = _threefry2x32(np.uint32(0), np.uint32(42), c0, c1)
    return (y0 ^ y1).reshape(_B, _V)


def _body(probs_ref, bits_ref, p_ref, s_ref):
    x = probs_ref[...]
    m = jnp.log(jnp.max(x, axis=-1, keepdims=True)) / _TEMP
    e = jnp.exp(jnp.log(x) / _TEMP - m)
    s1 = jnp.sum(e, axis=-1, keepdims=True)
    target = _TOP_P * s1

    ones = jnp.ones((_V, 1), jnp.float32)

    def masked_sum(t):
        return jnp.dot(
            jnp.where(e > t, e, 0.0), ones, preferred_element_type=jnp.float32
        )

    def step(_, carry):
        lo, hi = carry
        w = hi - lo
        mids = [lo + (w * k) // _SEARCH_K for k in range(1, _SEARCH_K)]
        preds = [
            masked_sum(lax.bitcast_convert_type(m, jnp.float32)) <= target
            for m in mids
        ]
        nhi = hi
        nlo = mids[-1] + 1
        for m, mprev, p in zip(
            reversed(mids), reversed([lo - 1] + mids[:-1]), reversed(preds)
        ):
            nhi = jnp.where(p, m, nhi)
            nlo = jnp.where(p, mprev + 1, nlo)
        return nlo, nhi

    lo0 = lax.bitcast_convert_type(_VSTAR_LB * s1, jnp.int32)
    hi0 = jnp.full((_ROWS, 1), _ONE_BITS, jnp.int32)
    _, hi = lax.fori_loop(0, _SEARCH_STEPS, step, (lo0, hi0))
    vstar = lax.bitcast_convert_type(hi, jnp.float32)

    keep = e >= vstar
    masked = jnp.where(keep, e, 0.0)
    s2 = jnp.sum(masked, axis=-1, keepdims=True)
    p_ref[...] = masked / s2

    bits = bits_ref[...]
    fb = (bits >> jnp.uint32(9)) | jnp.uint32(_ONE_BITS)
    u = lax.bitcast_convert_type(fb, jnp.float32) - jnp.float32(1.0)
    u = u * (jnp.float32(1.0) - _TINY) + _TINY
    u = jnp.maximum(_TINY, u)
    g = -jnp.log(-jnp.log(u))
    z = jnp.where(keep, g + jnp.log(e), jnp.float32(-3e38))
    s_ref[...] = jnp.argmax(z, axis=-1, keepdims=True).astype(jnp.int32)


def kernel(probs):
    bits = jnp.asarray(_gumbel_bits_const())
    p, samples = pl.pallas_call(
        _body,
        grid=(_B // _ROWS,),
        in_specs=[
            pl.BlockSpec((_ROWS, _V), lambda i: (i, 0)),
            pl.BlockSpec((_ROWS, _V), lambda i: (i, 0)),
        ],
        out_specs=[
            pl.BlockSpec((_ROWS, _V), lambda i: (i, 0)),
            pl.BlockSpec((_ROWS, 1), lambda i: (i, 0)),
        ],
        out_shape=[
            jax.ShapeDtypeStruct((_B, _V), jnp.float32),
            jax.ShapeDtypeStruct((_B, 1), jnp.int32),
        ],
    )(probs, bits)
    return p, samples.astype(jnp.int64)


# speedup vs baseline: 2.8158x; 2.8158x over previous
import functools

import numpy as np
import jax
import jax.numpy as jnp
from jax import lax
from jax.experimental import pallas as pl

_B, _V = 128, 100000
_ROWS = 16
_TEMP = np.float32(0.8)
_TOP_P = np.float32(0.9)
_ONE_BITS = 0x3F800000
_TINY = np.float32(np.finfo(np.float32).tiny)
_VSTAR_LB = np.float32(4.5e-7)
_SEARCH_K = 5
_SEARCH_STEPS = 12


def _threefry2x32(k1, k2, x0, x1):
    ks0 = np.uint32(k1)
    ks1 = np.uint32(k2)
    ks2 = np.uint32(ks0 ^ ks1 ^ np.uint32(0x1BD11BDA))
    rotations = [[13, 15, 26, 6], [17, 29, 16, 24]]
    ks = [ks0, ks1, ks2]
    x0 = (x0 + ks0).astype(np.uint32)
    x1 = (x1 + ks1).astype(np.uint32)
    for i in range(5):
        for r in rotations[i % 2]:
            x0 = (x0 + x1).astype(np.uint32)
            x1 = ((x1 << np.uint32(r)) | (x1 >> np.uint32(32 - r))).astype(np.uint32)
            x1 = (x1 ^ x0).astype(np.uint32)
        x0 = (x0 + ks[(i + 1) % 3]).astype(np.uint32)
        x1 = (x1 + ks[(i + 2) % 3] + np.uint32(i + 1)).astype(np.uint32)
    return x0, x1


@functools.cache
def _gumbel_bits_const():
    n = _B * _V
    c0 = np.zeros(n, dtype=np.uint32)
    c1 = np.arange(n, dtype=np.uint32)
    y0, y1 = _threefry2x32(np.uint32(0), np.uint32(42), c0, c1)
    return (y0 ^ y1).reshape(_B, _V)


def _body(probs_ref, bits_ref, p_ref, s_ref):
    x = probs_ref[...]
    m = jnp.log(jnp.max(x, axis=-1, keepdims=True)) / _TEMP
    e = jnp.exp(jnp.log(x) / _TEMP - m)
    s1 = jnp.sum(e, axis=-1, keepdims=True)
    target = _TOP_P * s1

    def masked_sum(t):
        return jnp.sum(jnp.where(e > t, e, 0.0), axis=-1, keepdims=True)

    def step(_, carry):
        lo, hi = carry
        w = hi - lo
        mids = [lo + (w * k) // _SEARCH_K for k in range(1, _SEARCH_K)]
        preds = [
            masked_sum(lax.bitcast_convert_type(m, jnp.float32)) <= target
            for m in mids
        ]
        nhi = hi
        nlo = mids[-1] + 1
        for m, mprev, p in zip(
            reversed(mids), reversed([lo - 1] + mids[:-1]), reversed(preds)
        ):
            nhi = jnp.where(p, m, nhi)
            nlo = jnp.where(p, mprev + 1, nlo)
        return nlo, nhi

    lo0 = lax.bitcast_convert_type(_VSTAR_LB * s1, jnp.int32)
    hi0 = jnp.full((_ROWS, 1), _ONE_BITS, jnp.int32)
    _, hi = lax.fori_loop(0, _SEARCH_STEPS, step, (lo0, hi0))
    vstar = lax.bitcast_convert_type(hi, jnp.float32)

    keep = e >= vstar
    masked = jnp.where(keep, e, 0.0)
    s2 = jnp.sum(masked, axis=-1, keepdims=True)
    p_ref[...] = masked / s2

    bits = bits_ref[...]
    fb = (bits >> jnp.uint32(9)) | jnp.uint32(_ONE_BITS)
    u = lax.bitcast_convert_type(fb, jnp.float32) - jnp.float32(1.0)
    u = u * (jnp.float32(1.0) - _TINY) + _TINY
    u = jnp.maximum(_TINY, u)
    g = -jnp.log(-jnp.log(u))
    z = jnp.where(keep, g + jnp.log(e), jnp.float32(-3e38))
    s_ref[...] = jnp.argmax(z, axis=-1, keepdims=True).astype(jnp.int32)


def kernel(probs):
    bits = jnp.asarray(_gumbel_bits_const())
    p, samples = pl.pallas_call(
        _body,
        grid=(_B // _ROWS,),
        in_specs=[
            pl.BlockSpec((_ROWS, _V), lambda i: (i, 0)),
            pl.BlockSpec((_ROWS, _V), lambda i: (i, 0)),
        ],
        out_specs=[
            pl.BlockSpec((_ROWS, _V), lambda i: (i, 0)),
            pl.BlockSpec((_ROWS, 1), lambda i: (i, 0)),
        ],
        out_shape=[
            jax.ShapeDtypeStruct((_B, _V), jnp.float32),
            jax.ShapeDtypeStruct((_B, 1), jnp.int32),
        ],
    )(probs, bits)
    return p, samples.astype(jnp.int64)


# speedup vs baseline: 2.9538x; 1.0490x over previous
import functools

import numpy as np
import jax
import jax.numpy as jnp
from jax import lax
from jax.experimental import pallas as pl

_B, _V = 128, 100000
_ROWS = 16
_TEMP = np.float32(0.8)
_TOP_P = np.float32(0.9)
_ONE_BITS = 0x3F800000
_TINY = np.float32(np.finfo(np.float32).tiny)
_VSTAR_LB = np.float32(4.5e-7)
_SEARCH_K = 3
_SEARCH_STEPS = 18


def _threefry2x32(k1, k2, x0, x1):
    ks0 = np.uint32(k1)
    ks1 = np.uint32(k2)
    ks2 = np.uint32(ks0 ^ ks1 ^ np.uint32(0x1BD11BDA))
    rotations = [[13, 15, 26, 6], [17, 29, 16, 24]]
    ks = [ks0, ks1, ks2]
    x0 = (x0 + ks0).astype(np.uint32)
    x1 = (x1 + ks1).astype(np.uint32)
    for i in range(5):
        for r in rotations[i % 2]:
            x0 = (x0 + x1).astype(np.uint32)
            x1 = ((x1 << np.uint32(r)) | (x1 >> np.uint32(32 - r))).astype(np.uint32)
            x1 = (x1 ^ x0).astype(np.uint32)
        x0 = (x0 + ks[(i + 1) % 3]).astype(np.uint32)
        x1 = (x1 + ks[(i + 2) % 3] + np.uint32(i + 1)).astype(np.uint32)
    return x0, x1


@functools.cache
def _gumbel_bits_const():
    n = _B * _V
    c0 = np.zeros(n, dtype=np.uint32)
    c1 = np.arange(n, dtype=np.uint32)
    y0, y1 = _threefry2x32(np.uint32(0), np.uint32(42), c0, c1)
    return (y0 ^ y1).reshape(_B, _V)


def _body(probs_ref, bits_ref, p_ref, s_ref):
    x = probs_ref[...]
    m = jnp.log(jnp.max(x, axis=-1, keepdims=True)) / _TEMP
    e = jnp.exp(jnp.log(x) / _TEMP - m)
    s1 = jnp.sum(e, axis=-1, keepdims=True)
    target = _TOP_P * s1

    def masked_sum(t):
        return jnp.sum(jnp.where(e > t, e, 0.0), axis=-1, keepdims=True)

    def step(_, carry):
        lo, hi = carry
        w = hi - lo
        mids = [lo + (w * k) // _SEARCH_K for k in range(1, _SEARCH_K)]
        preds = [
            masked_sum(lax.bitcast_convert_type(m, jnp.float32)) <= target
            for m in mids
        ]
        nhi = hi
        nlo = mids[-1] + 1
        for m, mprev, p in zip(
            reversed(mids), reversed([lo - 1] + mids[:-1]), reversed(preds)
        ):
            nhi = jnp.where(p, m, nhi)
            nlo = jnp.where(p, mprev + 1, nlo)
        return nlo, nhi

    lo0 = lax.bitcast_convert_type(_VSTAR_LB * s1, jnp.int32)
    hi0 = jnp.full((_ROWS, 1), _ONE_BITS, jnp.int32)
    _, hi = lax.fori_loop(0, _SEARCH_STEPS, step, (lo0, hi0))
    vstar = lax.bitcast_convert_type(hi, jnp.float32)

    keep = e >= vstar
    masked = jnp.where(keep, e, 0.0)
    s2 = jnp.sum(masked, axis=-1, keepdims=True)
    p_ref[...] = masked / s2

    bits = bits_ref[...]
    fb = (bits >> jnp.uint32(9)) | jnp.uint32(_ONE_BITS)
    u = lax.bitcast_convert_type(fb, jnp.float32) - jnp.float32(1.0)
    u = u * (jnp.float32(1.0) - _TINY) + _TINY
    u = jnp.maximum(_TINY, u)
    g = -jnp.log(-jnp.log(u))
    z = jnp.where(keep, g + jnp.log(e), jnp.float32(-3e38))
    s_ref[...] = jnp.argmax(z, axis=-1, keepdims=True).astype(jnp.int32)


def kernel(probs):
    bits = jnp.asarray(_gumbel_bits_const())
    p, samples = pl.pallas_call(
        _body,
        grid=(_B // _ROWS,),
        in_specs=[
            pl.BlockSpec((_ROWS, _V), lambda i: (i, 0)),
            pl.BlockSpec((_ROWS, _V), lambda i: (i, 0)),
        ],
        out_specs=[
            pl.BlockSpec((_ROWS, _V), lambda i: (i, 0)),
            pl.BlockSpec((_ROWS, 1), lambda i: (i, 0)),
        ],
        out_shape=[
            jax.ShapeDtypeStruct((_B, _V), jnp.float32),
            jax.ShapeDtypeStruct((_B, 1), jnp.int32),
        ],
    )(probs, bits)
    return p, samples.astype(jnp.int64)


# speedup vs baseline: 3.2185x; 1.0896x over previous
import functools

import numpy as np
import jax
import jax.numpy as jnp
from jax import lax
from jax.experimental import pallas as pl

_B, _V = 128, 100000
_ROWS = 16
_TEMP = np.float32(0.8)
_TOP_P = np.float32(0.9)
_ONE_BITS = 0x3F800000
_TINY = np.float32(np.finfo(np.float32).tiny)
_VSTAR_LB = np.float32(4.5e-7)
_SEARCH_K = 4
_SEARCH_STEPS = 14


def _threefry2x32(k1, k2, x0, x1):
    ks0 = np.uint32(k1)
    ks1 = np.uint32(k2)
    ks2 = np.uint32(ks0 ^ ks1 ^ np.uint32(0x1BD11BDA))
    rotations = [[13, 15, 26, 6], [17, 29, 16, 24]]
    ks = [ks0, ks1, ks2]
    x0 = (x0 + ks0).astype(np.uint32)
    x1 = (x1 + ks1).astype(np.uint32)
    for i in range(5):
        for r in rotations[i % 2]:
            x0 = (x0 + x1).astype(np.uint32)
            x1 = ((x1 << np.uint32(r)) | (x1 >> np.uint32(32 - r))).astype(np.uint32)
            x1 = (x1 ^ x0).astype(np.uint32)
        x0 = (x0 + ks[(i + 1) % 3]).astype(np.uint32)
        x1 = (x1 + ks[(i + 2) % 3] + np.uint32(i + 1)).astype(np.uint32)
    return x0, x1


@functools.cache
def _gumbel_bits_const():
    n = _B * _V
    c0 = np.zeros(n, dtype=np.uint32)
    c1 = np.arange(n, dtype=np.uint32)
    y0, y1 = _threefry2x32(np.uint32(0), np.uint32(42), c0, c1)
    return (y0 ^ y1).reshape(_B, _V)


def _body(probs_ref, bits_ref, p_ref, s_ref):
    x = probs_ref[...]
    lm = jnp.log(jnp.max(x, axis=-1, keepdims=True))
    s125 = (jnp.log(x) - lm) * jnp.float32(1.25)
    e = jnp.exp(s125)
    s1 = jnp.sum(e, axis=-1, keepdims=True)
    target = _TOP_P * s1

    def masked_sum(t):
        return jnp.sum(jnp.where(e > t, e, 0.0), axis=-1, keepdims=True)

    def step(_, carry):
        lo, hi = carry
        w = hi - lo
        mids = [lo + (w * k) // _SEARCH_K for k in range(1, _SEARCH_K)]
        preds = [
            masked_sum(lax.bitcast_convert_type(m, jnp.float32)) <= target
            for m in mids
        ]
        nhi = hi
        nlo = mids[-1] + 1
        for m, mprev, p in zip(
            reversed(mids), reversed([lo - 1] + mids[:-1]), reversed(preds)
        ):
            nhi = jnp.where(p, m, nhi)
            nlo = jnp.where(p, mprev + 1, nlo)
        return nlo, nhi

    lo0 = lax.bitcast_convert_type(_VSTAR_LB * s1, jnp.int32)
    hi0 = jnp.full((_ROWS, 1), _ONE_BITS, jnp.int32)
    _, hi = lax.fori_loop(0, _SEARCH_STEPS, step, (lo0, hi0))
    vstar = lax.bitcast_convert_type(hi, jnp.float32)

    keep = e >= vstar
    masked = jnp.where(keep, e, 0.0)
    s2 = jnp.sum(masked, axis=-1, keepdims=True)
    p_ref[...] = masked / s2

    bits = bits_ref[...]
    fb = (bits >> jnp.uint32(9)) | jnp.uint32(_ONE_BITS)
    u = lax.bitcast_convert_type(fb, jnp.float32) - jnp.float32(1.0)
    u = u * (jnp.float32(1.0) - _TINY) + _TINY
    u = jnp.maximum(_TINY, u)
    g = -jnp.log(-jnp.log(u))
    z = jnp.where(keep, g + s125, jnp.float32(-3e38))
    s_ref[...] = jnp.argmax(z, axis=-1, keepdims=True).astype(jnp.int32)


def kernel(probs):
    bits = jnp.asarray(_gumbel_bits_const())
    p, samples = pl.pallas_call(
        _body,
        grid=(_B // _ROWS,),
        in_specs=[
            pl.BlockSpec((_ROWS, _V), lambda i: (i, 0)),
            pl.BlockSpec((_ROWS, _V), lambda i: (i, 0)),
        ],
        out_specs=[
            pl.BlockSpec((_ROWS, _V), lambda i: (i, 0)),
            pl.BlockSpec((_ROWS, 1), lambda i: (i, 0)),
        ],
        out_shape=[
            jax.ShapeDtypeStruct((_B, _V), jnp.float32),
            jax.ShapeDtypeStruct((_B, 1), jnp.int32),
        ],
    )(probs, bits)
    return p, samples.astype(jnp.int64)
